# baseline (speedup 1.0000x reference)
import functools

import jax
import jax.numpy as jnp
from jax import lax
from jax.experimental import pallas as pl
from jax.experimental.pallas import tpu as pltpu
from jax.experimental.pallas import tpu_sc as plsc

B = 16384
L = 20
NBUCKETS = 100000
TPAD = 102400
NFEAT = 6

NC = 2
NS = 16
NW = NC * NS

CHUNK = 256
CPF = B // CHUNK
NCHUNKS = NFEAT * CPF
CPW = NCHUNKS // NW
GPC = CHUNK // 16


def _hash16(x):
    h = x.astype(jnp.uint32)
    h = h * jnp.uint32(2654435761)
    h = h ^ (h >> 16)
    h = h * jnp.uint32(2246822519)
    h = h ^ (h >> 13)
    return h % jnp.uint32(NBUCKETS)


def _sc_body(*refs):
    ids_refs = refs[0:NFEAT]
    w_refs = refs[NFEAT:2 * NFEAT]
    part_hbm = refs[2 * NFEAT]
    table_v, ids0_v, ids1_v, out_v, sem0, sem1 = refs[2 * NFEAT + 1:]
    bufs = (ids0_v, ids1_v)
    sems = (sem0, sem1)

    wid = lax.axis_index("c") * NS + lax.axis_index("s")
    c_lo = wid * CPW
    c_hi = c_lo + CPW

    def load_table(f):
        for i in range(NFEAT):
            @pl.when(f == i)
            def _load():
                pltpu.sync_copy(w_refs[i], table_v)

    def start_ids(c, b):
        f = c // CPF
        r0 = (c % CPF) * CHUNK
        for i in range(NFEAT):
            @pl.when(f == i)
            def _start():
                pltpu.async_copy(ids_refs[i].at[pl.ds(r0, CHUNK), :], bufs[b], sems[b])

    def wait_ids(b):
        pltpu.make_async_copy(ids_refs[0].at[pl.ds(0, CHUNK), :], bufs[b], sems[b]).wait()

    def compute_chunk(c, ids_v):
        f = c // CPF
        r0 = (c % CPF) * CHUNK

        def group(g, _):
            rows = g * 16 + lax.iota(jnp.int32, 16)
            hs = []
            acc = jnp.zeros((16,), jnp.float32)
            for j in range(L):
                idj = plsc.load_gather(ids_v, [rows, jnp.full((16,), j, jnp.int32)])
                h = _hash16(idj)
                hi = h.astype(jnp.int32)
                hw = hi >> 1
                word = plsc.load_gather(table_v, [hw >> 7, hw & 127])
                bits = jnp.where(hi & 1 != 0,
                                 word & jnp.int32(-65536),
                                 word << 16)
                wj = plsc.bitcast(bits, jnp.float32)
                if j == 0:
                    acc = wj
                else:
                    md = hs[0] ^ h
                    for k in range(1, j):
                        md = jnp.minimum(md, hs[k] ^ h)
                    acc = acc + jnp.where(md != 0, wj, 0.0)
                hs.append(h)
            out_v[pl.ds(g * 16, 16)] = acc
            return 0

        lax.fori_loop(0, GPC, group, 0)
        pltpu.sync_copy(out_v, part_hbm.at[f, pl.ds(r0, CHUNK)])

    f0 = c_lo // CPF
    f1 = (c_hi - 1) // CPF
    split = jnp.minimum(c_hi, (f0 + 1) * CPF)

    load_table(f0)
    start_ids(c_lo, 0)

    def chunk_step(t, _):
        @pl.when(t == split)
        def _swap_table():
            load_table(f1)

        for b in range(2):
            @pl.when(t % 2 == b)
            def _do():
                @pl.when(t + 1 < c_hi)
                def _prefetch():
                    start_ids(t + 1, 1 - b)
                wait_ids(b)
                compute_chunk(t, bufs[b])
        return 0

    lax.fori_loop(c_lo, c_hi, chunk_step, 0)


@jax.jit
def _sc_partials(*arrays):
    mesh = plsc.VectorSubcoreMesh(core_axis_name="c", subcore_axis_name="s")
    return pl.kernel(
        _sc_body,
        out_type=jax.ShapeDtypeStruct((NFEAT, B), jnp.float32),
        mesh=mesh,
        scratch_types=[
            pltpu.VMEM((TPAD // 256, 128), jnp.int32),
            pltpu.VMEM((CHUNK, L), jnp.int32),
            pltpu.VMEM((CHUNK, L), jnp.int32),
            pltpu.VMEM((CHUNK,), jnp.float32),
            pltpu.SemaphoreType.DMA,
            pltpu.SemaphoreType.DMA,
        ],
        compiler_params=pltpu.CompilerParams(needs_layout_passes=False),
    )(*arrays)


def _epilogue_body(part_ref, bias_ref, out_ref):
    out_ref[:, :] = jnp.sum(part_ref[:, :], axis=0, keepdims=True) + bias_ref[0, 0]


@jax.jit
def _epilogue(part, bias):
    out = pl.pallas_call(
        _epilogue_body,
        out_shape=jax.ShapeDtypeStruct((1, B), jnp.float32),
    )(part, bias.reshape(1, 1))
    return out.reshape(B, 1)


def kernel(user_id, item_id, category_id, shop_id, hist_item_id, target_item_id,
           w_user_id, w_item_id, w_category_id, w_shop_id, w_hist_item_id,
           w_target_item_id, bias):
    ids = [user_id, item_id, category_id, shop_id, hist_item_id, target_item_id]
    ids = [x.astype(jnp.int32) for x in ids]
    ws = [w_user_id, w_item_id, w_category_id, w_shop_id, w_hist_item_id,
          w_target_item_id]
    ws = [
        lax.bitcast_convert_type(
            jnp.pad(w, (0, TPAD - NBUCKETS)).astype(jnp.bfloat16).reshape(
                TPAD // 256, 128, 2),
            jnp.int32)
        for w in ws
    ]
    part = _sc_partials(*ids, *ws)
    return _epilogue(part, bias)


# speedup vs baseline: 2.1571x; 2.1571x over previous
import functools

import jax
import jax.numpy as jnp
from jax import lax
from jax.experimental import pallas as pl
from jax.experimental.pallas import tpu as pltpu
from jax.experimental.pallas import tpu_sc as plsc

B = 16384
L = 20
NBUCKETS = 100000
TROWS = 800
NFEAT = 6

NC = 2
NS = 16
NW = NC * NS

CHUNK = 256
CL = CHUNK * L
CROWS = CL // 128
CPF = B // CHUNK
NCHUNKS = NFEAT * CPF
CPW = NCHUNKS // NW
GPC = CHUNK // 16


def _hash16(x):
    h = x.astype(jnp.uint32)
    h = h * jnp.uint32(2654435761)
    h = h ^ (h >> 16)
    h = h * jnp.uint32(2246822519)
    h = h ^ (h >> 13)
    return h % jnp.uint32(NBUCKETS)


def _sc_body(*refs):
    ids_refs = refs[0:NFEAT]
    w_refs = refs[NFEAT:2 * NFEAT]
    part_hbm = refs[2 * NFEAT]
    table_v, ids0_v, ids1_v, out_v, sem0, sem1 = refs[2 * NFEAT + 1:]
    bufs = (ids0_v, ids1_v)
    sems = (sem0, sem1)

    wid = lax.axis_index("c") * NS + lax.axis_index("s")
    c_lo = wid * CPW
    c_hi = c_lo + CPW

    def load_table(f):
        for i in range(NFEAT):
            @pl.when(f == i)
            def _load():
                pltpu.sync_copy(w_refs[i], table_v)

    def start_ids(c, b):
        f = c // CPF
        row0 = (c % CPF) * CROWS
        for i in range(NFEAT):
            @pl.when(f == i)
            def _start():
                pltpu.async_copy(ids_refs[i].at[pl.ds(row0, CROWS), :], bufs[b], sems[b])

    def wait_ids(b):
        pltpu.make_async_copy(ids_refs[0].at[pl.ds(0, CROWS), :], bufs[b], sems[b]).wait()

    def compute_chunk(c, ids_v):
        f = c // CPF
        r0 = (c % CPF) * CHUNK

        def group(g, _):
            rows = g * 16 + lax.iota(jnp.int32, 16)
            base = rows * L
            hs = []
            acc = jnp.zeros((16,), jnp.float32)
            for j in range(L):
                fl = base + j
                idj = plsc.load_gather(ids_v, [fl >> 7, fl & 127])
                h = _hash16(idj)
                hi = h.astype(jnp.int32)
                wj = plsc.load_gather(table_v, [hi >> 7, hi & 127])
                if j == 0:
                    acc = wj
                else:
                    md = hs[0] ^ h
                    for k in range(1, j):
                        md = jnp.minimum(md, hs[k] ^ h)
                    acc = acc + jnp.where(md != 0, wj, 0.0)
                hs.append(h)
            out_v[pl.ds(g * 16, 16)] = acc
            return 0

        lax.fori_loop(0, GPC, group, 0)
        pltpu.sync_copy(out_v, part_hbm.at[f, pl.ds(r0, CHUNK)])

    f0 = c_lo // CPF
    f1 = (c_hi - 1) // CPF
    split = jnp.minimum(c_hi, (f0 + 1) * CPF)

    load_table(f0)
    start_ids(c_lo, 0)

    def chunk_step(t, _):
        @pl.when(t == split)
        def _swap_table():
            load_table(f1)

        for b in range(2):
            @pl.when(t % 2 == b)
            def _do():
                @pl.when(t + 1 < c_hi)
                def _prefetch():
                    start_ids(t + 1, 1 - b)
                wait_ids(b)
                compute_chunk(t, bufs[b])
        return 0

    lax.fori_loop(c_lo, c_hi, chunk_step, 0)


@jax.jit
def _sc_partials(*arrays):
    mesh = plsc.VectorSubcoreMesh(core_axis_name="c", subcore_axis_name="s")
    return pl.kernel(
        _sc_body,
        out_type=jax.ShapeDtypeStruct((NFEAT, B), jnp.float32),
        mesh=mesh,
        scratch_types=[
            pltpu.VMEM((TROWS, 128), jnp.float32),
            pltpu.VMEM((CROWS, 128), jnp.int32),
            pltpu.VMEM((CROWS, 128), jnp.int32),
            pltpu.VMEM((CHUNK,), jnp.float32),
            pltpu.SemaphoreType.DMA,
            pltpu.SemaphoreType.DMA,
        ],
        compiler_params=pltpu.CompilerParams(needs_layout_passes=False),
    )(*arrays)


def _epilogue_body(part_ref, bias_ref, out_ref):
    out_ref[:, :] = jnp.sum(part_ref[:, :], axis=0, keepdims=True) + bias_ref[0, 0]


@jax.jit
def _epilogue(part, bias):
    out = pl.pallas_call(
        _epilogue_body,
        out_shape=jax.ShapeDtypeStruct((1, B), jnp.float32),
    )(part, bias.reshape(1, 1))
    return out.reshape(B, 1)


def kernel(user_id, item_id, category_id, shop_id, hist_item_id, target_item_id,
           w_user_id, w_item_id, w_category_id, w_shop_id, w_hist_item_id,
           w_target_item_id, bias):
    ids = [user_id, item_id, category_id, shop_id, hist_item_id, target_item_id]
    ids = [x.astype(jnp.int32).reshape(B * L // 128, 128) for x in ids]
    ws = [w_user_id, w_item_id, w_category_id, w_shop_id, w_hist_item_id,
          w_target_item_id]
    ws = [jnp.pad(w, (0, TROWS * 128 - NBUCKETS)).reshape(TROWS, 128) for w in ws]
    part = _sc_partials(*ids, *ws)
    return _epilogue(part, bias)


# speedup vs baseline: 2.3341x; 1.0821x over previous
import functools

import jax
import jax.numpy as jnp
from jax import lax
from jax.experimental import pallas as pl
from jax.experimental.pallas import tpu as pltpu
from jax.experimental.pallas import tpu_sc as plsc

B = 16384
L = 20
NBUCKETS = 100000
TROWS = 800
NFEAT = 6

NC = 2
NS = 16
NW = NC * NS

CHUNK = 64
CPF = B // CHUNK
NCHUNKS = NFEAT * CPF
CPW = NCHUNKS // NW
GPC = CHUNK // 16


def _hash16(x):
    h = x.astype(jnp.uint32)
    h = h * jnp.uint32(2654435761)
    h = h ^ (h >> 16)
    h = h * jnp.uint32(2246822519)
    h = h ^ (h >> 13)
    return h % jnp.uint32(NBUCKETS)


def _sc_body(*refs):
    ids_refs = refs[0:NFEAT]
    w_refs = refs[NFEAT:2 * NFEAT]
    part_hbm = refs[2 * NFEAT]
    table_v, ids0_v, ids1_v, out_v, sem0, sem1 = refs[2 * NFEAT + 1:]
    bufs = (ids0_v, ids1_v)
    sems = (sem0, sem1)

    wid = lax.axis_index("c") * NS + lax.axis_index("s")
    c_lo = wid * CPW
    c_hi = c_lo + CPW

    def load_table(f):
        for i in range(NFEAT):
            @pl.when(f == i)
            def _load():
                pltpu.sync_copy(w_refs[i], table_v)

    def start_ids(c, b):
        f = c // CPF
        r0 = (c % CPF) * CHUNK
        for i in range(NFEAT):
            @pl.when(f == i)
            def _start():
                pltpu.async_copy(ids_refs[i].at[pl.ds(r0, CHUNK), :], bufs[b], sems[b])

    def wait_ids(b):
        pltpu.make_async_copy(ids_refs[0].at[pl.ds(0, CHUNK), :], bufs[b], sems[b]).wait()

    def compute_chunk(c, ids_v):
        f = c // CPF
        r0 = (c % CPF) * CHUNK

        def group(g, _):
            rows = g * 16 + lax.iota(jnp.int32, 16)
            hs = []
            acc = jnp.zeros((16,), jnp.float32)
            for j in range(L):
                idj = plsc.load_gather(ids_v, [rows, jnp.full((16,), j, jnp.int32)])
                h = _hash16(idj)
                hi = h.astype(jnp.int32)
                wj = plsc.load_gather(table_v, [hi >> 7, hi & 127])
                if j == 0:
                    acc = wj
                else:
                    md = hs[0] ^ h
                    for k in range(1, j):
                        md = jnp.minimum(md, hs[k] ^ h)
                    acc = acc + jnp.where(md != 0, wj, 0.0)
                hs.append(h)
            out_v[pl.ds(g * 16, 16)] = acc
            return 0

        lax.fori_loop(0, GPC, group, 0)
        pltpu.sync_copy(out_v, part_hbm.at[f, pl.ds(r0, CHUNK)])

    f0 = c_lo // CPF
    f1 = (c_hi - 1) // CPF
    split = jnp.minimum(c_hi, (f0 + 1) * CPF)

    load_table(f0)
    start_ids(c_lo, 0)

    def chunk_step(t, _):
        @pl.when(t == split)
        def _swap_table():
            load_table(f1)

        for b in range(2):
            @pl.when(t % 2 == b)
            def _do():
                @pl.when(t + 1 < c_hi)
                def _prefetch():
                    start_ids(t + 1, 1 - b)
                wait_ids(b)
                compute_chunk(t, bufs[b])
        return 0

    lax.fori_loop(c_lo, c_hi, chunk_step, 0)


@jax.jit
def _sc_partials(*arrays):
    mesh = plsc.VectorSubcoreMesh(core_axis_name="c", subcore_axis_name="s")
    return pl.kernel(
        _sc_body,
        out_type=jax.ShapeDtypeStruct((NFEAT, B), jnp.float32),
        mesh=mesh,
        scratch_types=[
            pltpu.VMEM((TROWS, 128), jnp.float32),
            pltpu.VMEM((CHUNK, L), jnp.int32),
            pltpu.VMEM((CHUNK, L), jnp.int32),
            pltpu.VMEM((CHUNK,), jnp.float32),
            pltpu.SemaphoreType.DMA,
            pltpu.SemaphoreType.DMA,
        ],
        compiler_params=pltpu.CompilerParams(needs_layout_passes=False),
    )(*arrays)


def _epilogue_body(part_ref, bias_ref, out_ref):
    out_ref[:, :] = jnp.sum(part_ref[:, :], axis=0, keepdims=True) + bias_ref[0, 0]


@jax.jit
def _epilogue(part, bias):
    out = pl.pallas_call(
        _epilogue_body,
        out_shape=jax.ShapeDtypeStruct((1, B), jnp.float32),
    )(part, bias.reshape(1, 1))
    return out.reshape(B, 1)


def kernel(user_id, item_id, category_id, shop_id, hist_item_id, target_item_id,
           w_user_id, w_item_id, w_category_id, w_shop_id, w_hist_item_id,
           w_target_item_id, bias):
    ids = [user_id, item_id, category_id, shop_id, hist_item_id, target_item_id]
    ids = [x.astype(jnp.int32) for x in ids]
    ws = [w_user_id, w_item_id, w_category_id, w_shop_id, w_hist_item_id,
          w_target_item_id]
    ws = [jnp.pad(w, (0, TROWS * 128 - NBUCKETS)).reshape(TROWS, 128) for w in ws]
    part = _sc_partials(*ids, *ws)
    return _epilogue(part, bias)


# speedup vs baseline: 2.3901x; 1.0240x over previous
import functools

import jax
import jax.numpy as jnp
from jax import lax
from jax.experimental import pallas as pl
from jax.experimental.pallas import tpu as pltpu
from jax.experimental.pallas import tpu_sc as plsc

B = 16384
L = 20
NBUCKETS = 100000
TROWS = 800
NFEAT = 6

NC = 2
NS = 16
NW = NC * NS

CHUNK = 64
CPF = B // CHUNK
NCHUNKS = NFEAT * CPF
CPW = NCHUNKS // NW
GPC = CHUNK // 16


def _hash16(x):
    h = x.astype(jnp.uint32)
    h = h * jnp.uint32(2654435761)
    h = h ^ (h >> 16)
    h = h * jnp.uint32(2246822519)
    h = h ^ (h >> 13)
    return h % jnp.uint32(NBUCKETS)


def _sc_body(*refs):
    ids_refs = refs[0:NFEAT]
    w_refs = refs[NFEAT:2 * NFEAT]
    part_hbm = refs[2 * NFEAT]
    (table_v, ids0_v, ids1_v, out0_v, out1_v,
     sem0, sem1, semo0, semo1) = refs[2 * NFEAT + 1:]
    bufs = (ids0_v, ids1_v)
    sems = (sem0, sem1)
    obufs = (out0_v, out1_v)
    osems = (semo0, semo1)

    wid = lax.axis_index("c") * NS + lax.axis_index("s")
    c_lo = wid * CPW
    c_hi = c_lo + CPW

    def load_table(f):
        for i in range(NFEAT):
            @pl.when(f == i)
            def _load():
                pltpu.sync_copy(w_refs[i], table_v)

    def start_ids(c, b):
        f = c // CPF
        r0 = (c % CPF) * CHUNK
        for i in range(NFEAT):
            @pl.when(f == i)
            def _start():
                pltpu.async_copy(ids_refs[i].at[pl.ds(r0, CHUNK), :], bufs[b], sems[b])

    def wait_ids(b):
        pltpu.make_async_copy(ids_refs[0].at[pl.ds(0, CHUNK), :], bufs[b], sems[b]).wait()

    def wait_out(b):
        pltpu.make_async_copy(obufs[b], part_hbm.at[0, pl.ds(0, CHUNK)], osems[b]).wait()

    def compute_chunk(c, ids_v, out_v, osem):
        f = c // CPF
        r0 = (c % CPF) * CHUNK

        def group(g, _):
            rows = g * 16 + lax.iota(jnp.int32, 16)
            hs = []
            acc = jnp.zeros((16,), jnp.float32)
            for j in range(L):
                idj = plsc.load_gather(ids_v, [rows, jnp.full((16,), j, jnp.int32)])
                h = _hash16(idj)
                hi = h.astype(jnp.int32)
                wj = plsc.load_gather(table_v, [hi >> 7, hi & 127])
                if j == 0:
                    acc = wj
                else:
                    md = hs[0] ^ h
                    for k in range(1, j):
                        md = jnp.minimum(md, hs[k] ^ h)
                    acc = acc + jnp.where(md != 0, wj, 0.0)
                hs.append(h)
            out_v[pl.ds(g * 16, 16)] = acc
            return 0

        lax.fori_loop(0, GPC, group, 0)
        pltpu.async_copy(out_v, part_hbm.at[f, pl.ds(r0, CHUNK)], osem)

    f0 = c_lo // CPF
    f1 = (c_hi - 1) // CPF
    split = jnp.minimum(c_hi, (f0 + 1) * CPF)

    load_table(f0)
    start_ids(c_lo, 0)

    def chunk_step(t, _):
        @pl.when(t == split)
        def _swap_table():
            load_table(f1)

        for b in range(2):
            @pl.when(t % 2 == b)
            def _do():
                @pl.when(t + 1 < c_hi)
                def _prefetch():
                    start_ids(t + 1, 1 - b)
                wait_ids(b)

                @pl.when(t - c_lo >= 2)
                def _drain_out():
                    wait_out(b)
                compute_chunk(t, bufs[b], obufs[b], osems[b])
        return 0

    lax.fori_loop(c_lo, c_hi, chunk_step, 0)
    wait_out(0)
    wait_out(1)


@jax.jit
def _sc_partials(*arrays):
    mesh = plsc.VectorSubcoreMesh(core_axis_name="c", subcore_axis_name="s")
    return pl.kernel(
        _sc_body,
        out_type=jax.ShapeDtypeStruct((NFEAT, B), jnp.float32),
        mesh=mesh,
        scratch_types=[
            pltpu.VMEM((TROWS, 128), jnp.float32),
            pltpu.VMEM((CHUNK, L), jnp.int32),
            pltpu.VMEM((CHUNK, L), jnp.int32),
            pltpu.VMEM((CHUNK,), jnp.float32),
            pltpu.VMEM((CHUNK,), jnp.float32),
            pltpu.SemaphoreType.DMA,
            pltpu.SemaphoreType.DMA,
            pltpu.SemaphoreType.DMA,
            pltpu.SemaphoreType.DMA,
        ],
        compiler_params=pltpu.CompilerParams(needs_layout_passes=False),
    )(*arrays)


def _epilogue_body(part_ref, bias_ref, out_ref):
    out_ref[:, :] = jnp.sum(part_ref[:, :], axis=0, keepdims=True) + bias_ref[0, 0]


@jax.jit
def _epilogue(part, bias):
    out = pl.pallas_call(
        _epilogue_body,
        out_shape=jax.ShapeDtypeStruct((1, B), jnp.float32),
    )(part, bias.reshape(1, 1))
    return out.reshape(B, 1)


def kernel(user_id, item_id, category_id, shop_id, hist_item_id, target_item_id,
           w_user_id, w_item_id, w_category_id, w_shop_id, w_hist_item_id,
           w_target_item_id, bias):
    ids = [user_id, item_id, category_id, shop_id, hist_item_id, target_item_id]
    ids = [x.astype(jnp.int32) for x in ids]
    ws = [w_user_id, w_item_id, w_category_id, w_shop_id, w_hist_item_id,
          w_target_item_id]
    ws = [jnp.pad(w, (0, TROWS * 128 - NBUCKETS)).reshape(TROWS, 128) for w in ws]
    part = _sc_partials(*ids, *ws)
    return _epilogue(part, bias)


# speedup vs baseline: 2.7329x; 1.1434x over previous
import functools

import jax
import jax.numpy as jnp
from jax import lax
from jax.experimental import pallas as pl
from jax.experimental.pallas import tpu as pltpu
from jax.experimental.pallas import tpu_sc as plsc

B = 16384
L = 20
NBUCKETS = 100000
TROWS = 800
NFEAT = 6

NC = 2
NS = 16
NW = NC * NS

CHUNK = 64
CPF = B // CHUNK
NCHUNKS = NFEAT * CPF
CPW = NCHUNKS // NW
GPC = CHUNK // 16


def _hash16(x):
    h = x.astype(jnp.uint32)
    h = h * jnp.uint32(2654435761)
    h = h ^ (h >> 16)
    h = h * jnp.uint32(2246822519)
    h = h ^ (h >> 13)
    return h % jnp.uint32(NBUCKETS)


def _sc_body(*refs):
    ids_refs = refs[0:NFEAT]
    w_refs = refs[NFEAT:2 * NFEAT]
    part_hbm = refs[2 * NFEAT]
    (table_v, ids0_v, ids1_v, out0_v, out1_v,
     sem0, sem1, semo0, semo1) = refs[2 * NFEAT + 1:]
    bufs = (ids0_v, ids1_v)
    sems = (sem0, sem1)
    obufs = (out0_v, out1_v)
    osems = (semo0, semo1)

    wid = lax.axis_index("c") * NS + lax.axis_index("s")
    c_lo = wid * CPW
    c_hi = c_lo + CPW

    def load_table(f):
        for i in range(NFEAT):
            @pl.when(f == i)
            def _load():
                pltpu.sync_copy(w_refs[i], table_v)

    def start_ids(c, b):
        f = c // CPF
        r0 = (c % CPF) * CHUNK
        for i in range(NFEAT):
            @pl.when(f == i)
            def _start():
                pltpu.async_copy(ids_refs[i].at[pl.ds(r0, CHUNK), :], bufs[b], sems[b])

    def wait_ids(b):
        pltpu.make_async_copy(ids_refs[0].at[pl.ds(0, CHUNK), :], bufs[b], sems[b]).wait()

    def wait_out(b):
        pltpu.make_async_copy(obufs[b], part_hbm.at[0, pl.ds(0, CHUNK)], osems[b]).wait()

    def compute_chunk(c, ids_v, out_v, osem):
        f = c // CPF
        r0 = (c % CPF) * CHUNK

        def group(g, _):
            rows = g * 16 + lax.iota(jnp.int32, 16)
            hs = []
            acc = jnp.zeros((16,), jnp.float32)
            for j in range(L):
                idj = plsc.load_gather(ids_v, [rows, jnp.full((16,), j, jnp.int32)])
                h = _hash16(idj)
                hi = h.astype(jnp.int32)
                wj = plsc.load_gather(table_v, [hi >> 7, hi & 127])
                if j == 0:
                    acc = wj
                else:
                    md = hs[0] ^ h
                    for k in range(1, j):
                        md = jnp.minimum(md, hs[k] ^ h)
                    acc = acc + jnp.where(md != 0, wj, 0.0)
                hs.append(h)
            out_v[pl.ds(g * 16, 16)] = acc
            return 0

        for g in range(GPC):
            group(g, 0)
        pltpu.async_copy(out_v, part_hbm.at[f, pl.ds(r0, CHUNK)], osem)

    f0 = c_lo // CPF
    f1 = (c_hi - 1) // CPF
    split = jnp.minimum(c_hi, (f0 + 1) * CPF)

    load_table(f0)
    start_ids(c_lo, 0)

    def chunk_step(t, _):
        @pl.when(t == split)
        def _swap_table():
            load_table(f1)

        for b in range(2):
            @pl.when(t % 2 == b)
            def _do():
                @pl.when(t + 1 < c_hi)
                def _prefetch():
                    start_ids(t + 1, 1 - b)
                wait_ids(b)

                @pl.when(t - c_lo >= 2)
                def _drain_out():
                    wait_out(b)
                compute_chunk(t, bufs[b], obufs[b], osems[b])
        return 0

    lax.fori_loop(c_lo, c_hi, chunk_step, 0)
    wait_out(0)
    wait_out(1)


@jax.jit
def _sc_partials(*arrays):
    mesh = plsc.VectorSubcoreMesh(core_axis_name="c", subcore_axis_name="s")
    return pl.kernel(
        _sc_body,
        out_type=jax.ShapeDtypeStruct((NFEAT, B), jnp.float32),
        mesh=mesh,
        scratch_types=[
            pltpu.VMEM((TROWS, 128), jnp.float32),
            pltpu.VMEM((CHUNK, L), jnp.int32),
            pltpu.VMEM((CHUNK, L), jnp.int32),
            pltpu.VMEM((CHUNK,), jnp.float32),
            pltpu.VMEM((CHUNK,), jnp.float32),
            pltpu.SemaphoreType.DMA,
            pltpu.SemaphoreType.DMA,
            pltpu.SemaphoreType.DMA,
            pltpu.SemaphoreType.DMA,
        ],
        compiler_params=pltpu.CompilerParams(needs_layout_passes=False),
    )(*arrays)


def _epilogue_body(part_ref, bias_ref, out_ref):
    out_ref[:, :] = jnp.sum(part_ref[:, :], axis=0, keepdims=True) + bias_ref[0, 0]


@jax.jit
def _epilogue(part, bias):
    out = pl.pallas_call(
        _epilogue_body,
        out_shape=jax.ShapeDtypeStruct((1, B), jnp.float32),
    )(part, bias.reshape(1, 1))
    return out.reshape(B, 1)


def kernel(user_id, item_id, category_id, shop_id, hist_item_id, target_item_id,
           w_user_id, w_item_id, w_category_id, w_shop_id, w_hist_item_id,
           w_target_item_id, bias):
    ids = [user_id, item_id, category_id, shop_id, hist_item_id, target_item_id]
    ids = [x.astype(jnp.int32) for x in ids]
    ws = [w_user_id, w_item_id, w_category_id, w_shop_id, w_hist_item_id,
          w_target_item_id]
    ws = [jnp.pad(w, (0, TROWS * 128 - NBUCKETS)).reshape(TROWS, 128) for w in ws]
    part = _sc_partials(*ids, *ws)
    return _epilogue(part, bias)


# speedup vs baseline: 2.7404x; 1.0027x over previous
import jax
import jax.numpy as jnp
from jax import lax
from jax.experimental import pallas as pl
from jax.experimental.pallas import tpu as pltpu
from jax.experimental.pallas import tpu_sc as plsc

B = 16384
L = 20
NBUCKETS = 100000
TROWS = 800
NFEAT = 6

NC = 2
NS = 16
NW = NC * NS

CHUNK = 64
CPF = B // CHUNK
NCHUNKS = NFEAT * CPF
CPW = NCHUNKS // NW
GPC = CHUNK // 16


def _hash16(x):
    h = x.astype(jnp.uint32)
    h = h * jnp.uint32(2654435761)
    h = h ^ (h >> 16)
    h = h * jnp.uint32(2246822519)
    h = h ^ (h >> 13)
    return h % jnp.uint32(NBUCKETS)


def _sc_body(*refs):
    ids_refs = refs[0:NFEAT]
    w_refs = refs[NFEAT:2 * NFEAT]
    part_hbm = refs[2 * NFEAT]
    (table_v, ids0_v, ids1_v, out0_v, out1_v,
     sem0, sem1, semo0, semo1) = refs[2 * NFEAT + 1:]
    bufs = (ids0_v, ids1_v)
    sems = (sem0, sem1)
    obufs = (out0_v, out1_v)
    osems = (semo0, semo1)

    wid = lax.axis_index("c") * NS + lax.axis_index("s")
    c_lo = wid * CPW
    c_hi = c_lo + CPW

    def load_table(f):
        for i in range(NFEAT):
            @pl.when(f == i)
            def _load():
                pltpu.sync_copy(w_refs[i], table_v)

    def start_ids(c, b):
        f = c // CPF
        r0 = (c % CPF) * CHUNK
        for i in range(NFEAT):
            @pl.when(f == i)
            def _start():
                pltpu.async_copy(ids_refs[i].at[pl.ds(r0, CHUNK), :], bufs[b], sems[b])

    def wait_ids(b):
        pltpu.make_async_copy(ids_refs[0].at[pl.ds(0, CHUNK), :], bufs[b], sems[b]).wait()

    def wait_out(b):
        pltpu.make_async_copy(obufs[b], part_hbm.at[0, pl.ds(0, CHUNK)], osems[b]).wait()

    def compute_chunk(c, ids_v, out_v, osem):
        f = c // CPF
        r0 = (c % CPF) * CHUNK

        def group(g, _):
            rows = g * 16 + lax.iota(jnp.int32, 16)
            hs = []
            acc = jnp.zeros((16,), jnp.float32)
            for j in range(L):
                idj = plsc.load_gather(ids_v, [rows, jnp.full((16,), j, jnp.int32)])
                h = _hash16(idj)
                hi = h.astype(jnp.int32)
                wj = plsc.load_gather(table_v, [hi >> 7, hi & 127])
                if j == 0:
                    acc = wj
                else:
                    md = hs[0] ^ h
                    for k in range(1, j):
                        md = jnp.minimum(md, hs[k] ^ h)
                    acc = acc + jnp.where(md != 0, wj, 0.0)
                hs.append(h)
            out_v[pl.ds(g * 16, 16)] = acc
            return 0

        for g in range(GPC):
            group(g, 0)
        pltpu.async_copy(out_v, part_hbm.at[f, pl.ds(r0, CHUNK)], osem)

    f0 = c_lo // CPF
    f1 = (c_hi - 1) // CPF
    split = jnp.minimum(c_hi, (f0 + 1) * CPF)

    load_table(f0)
    start_ids(c_lo, 0)

    def chunk_step(t, _):
        @pl.when(t == split)
        def _swap_table():
            load_table(f1)

        for b in range(2):
            @pl.when(t % 2 == b)
            def _do():
                @pl.when(t + 1 < c_hi)
                def _prefetch():
                    start_ids(t + 1, 1 - b)
                wait_ids(b)

                @pl.when(t - c_lo >= 2)
                def _drain_out():
                    wait_out(b)
                compute_chunk(t, bufs[b], obufs[b], osems[b])
        return 0

    lax.fori_loop(c_lo, c_hi, chunk_step, 0)
    wait_out(0)
    wait_out(1)


@jax.jit
def _sc_partials(*arrays):
    mesh = plsc.VectorSubcoreMesh(core_axis_name="c", subcore_axis_name="s")
    return pl.kernel(
        _sc_body,
        out_type=jax.ShapeDtypeStruct((NFEAT, B), jnp.float32),
        mesh=mesh,
        scratch_types=[
            pltpu.VMEM((TROWS, 128), jnp.float32),
            pltpu.VMEM((CHUNK, L), jnp.int32),
            pltpu.VMEM((CHUNK, L), jnp.int32),
            pltpu.VMEM((CHUNK,), jnp.float32),
            pltpu.VMEM((CHUNK,), jnp.float32),
            pltpu.SemaphoreType.DMA,
            pltpu.SemaphoreType.DMA,
            pltpu.SemaphoreType.DMA,
            pltpu.SemaphoreType.DMA,
        ],
        compiler_params=pltpu.CompilerParams(needs_layout_passes=False),
    )(*arrays)


def _epilogue_body(part_ref, bias_ref, out_ref):
    out_ref[:, :] = jnp.sum(part_ref[:, :], axis=0, keepdims=True) + bias_ref[0, 0]


@jax.jit
def _epilogue(part, bias):
    out = pl.pallas_call(
        _epilogue_body,
        out_shape=jax.ShapeDtypeStruct((1, B), jnp.float32),
    )(part, bias.reshape(1, 1))
    return out.reshape(B, 1)


def kernel(user_id, item_id, category_id, shop_id, hist_item_id, target_item_id,
           w_user_id, w_item_id, w_category_id, w_shop_id, w_hist_item_id,
           w_target_item_id, bias):
    ids = [user_id, item_id, category_id, shop_id, hist_item_id, target_item_id]
    ids = [x.astype(jnp.int32) for x in ids]
    ws = [w_user_id, w_item_id, w_category_id, w_shop_id, w_hist_item_id,
          w_target_item_id]
    ws = [jnp.pad(w, (0, TROWS * 128 - NBUCKETS)).reshape(TROWS, 128) for w in ws]
    part = _sc_partials(*ids, *ws)
    return _epilogue(part, bias)
